# baseline (speedup 1.0000x reference)
import functools

import jax
import jax.numpy as jnp
from jax import lax
from jax.experimental import pallas as pl
from jax.experimental.pallas import tpu as pltpu
from jax.experimental.pallas import tpu_sc as plsc

_N = 10000
_E = 320000
_D = 128
_NC = 2
_NS = 16
_NW = _NC * _NS
_EPW = _E // _NW
_CH = 80
_NCHUNK = _EPW // _CH
_NPAD = 10112
_RPT = _NPAD // _NS
_VPR = _D // 16


def _sc_round(x, e, idx4, zrows):
    with_e = e is not None
    mesh = plsc.VectorSubcoreMesh(
        core_axis_name="c", subcore_axis_name="s",
        num_cores=_NC, num_subcores=_NS)

    scratch = [
        [pltpu.VMEM((2, _CH), jnp.int32)] * 3,
        [pltpu.VMEM((_CH, _D), jnp.float32)] * 2,
        pltpu.VMEM_SHARED((_NPAD, _D), jnp.float32),
        [pltpu.SemaphoreType.DMA] * 3,
        [pltpu.SemaphoreType.DMA] * 2,
    ]
    if with_e:
        scratch.insert(2, [pltpu.VMEM((_CH, _D), jnp.float32)] * 2)
        scratch.append([pltpu.SemaphoreType.DMA] * 2)

    def body(*refs):
        if with_e:
            (x_hbm, e_hbm, idx_hbm, z_hbm, out_hbm,
             ibuf, rbuf, ebuf, acc, isem, gsem, esem) = refs
        else:
            (x_hbm, idx_hbm, z_hbm, out_hbm,
             ibuf, rbuf, acc, isem, gsem) = refs
            e_hbm = ebuf = esem = None
        c = lax.axis_index("c")
        s = lax.axis_index("s")
        wid = c * _NS + s

        pltpu.sync_copy(z_hbm, acc.at[pl.ds(s * _RPT, _RPT)])
        plsc.subcore_barrier()

        def start_idx(j, q):
            pltpu.async_copy(idx_hbm.at[wid, j], ibuf[q % 3], isem[q % 3])

        def wait_idx(q):
            pltpu.make_async_copy(
                idx_hbm.at[wid, 0], ibuf[q % 3], isem[q % 3]).wait()

        def start_gather(j, q):
            pltpu.async_copy(
                x_hbm.at[ibuf[q % 3].at[0]], rbuf[q % 2], gsem[q % 2])
            if with_e:
                off = wid * _EPW + j * _CH
                pltpu.async_copy(
                    e_hbm.at[pl.ds(off, _CH)], ebuf[q % 2], esem[q % 2])

        def step(j, q, start_next=True, start_idx2=True):
            if start_next:
                wait_idx(q + 1)
                start_gather(j + 1, q + 1)
            if start_idx2:
                start_idx(j + 2, q + 2)
            pltpu.make_async_copy(
                x_hbm.at[ibuf[q % 3].at[0]], rbuf[q % 2],
                gsem[q % 2]).wait()
            if with_e:
                pltpu.make_async_copy(
                    e_hbm.at[pl.ds(0, _CH)], ebuf[q % 2],
                    esem[q % 2]).wait()

                @plsc.parallel_loop(0, _CH, unroll=4)
                def rloop(i):
                    for jj in range(_VPR):
                        sl = pl.ds(jj * 16, 16)
                        v = rbuf[q % 2][i, sl] + ebuf[q % 2][i, sl]
                        rbuf[q % 2][i, sl] = jnp.maximum(v, 0.0)
            pltpu.sync_copy(rbuf[q % 2], acc.at[ibuf[q % 3].at[1]],
                            add=True)

        pltpu.sync_copy(idx_hbm.at[wid, 0], ibuf[0])
        start_gather(0, 0)
        start_idx(1, 1)
        step(0, 0)
        step(1, 1)

        def group(k, carry):
            j0 = 6 * k + 2
            for q in range(6):
                step(j0 + q, (2 + q) % 6)
            return carry
        lax.fori_loop(0, 20, group, 0)

        step(122, 2)
        step(123, 3, start_idx2=False)
        step(124, 4, start_next=False, start_idx2=False)
        plsc.subcore_barrier()

        pltpu.sync_copy(acc.at[pl.ds(s * _RPT, _RPT)],
                        out_hbm.at[c, pl.ds(s * _RPT, _RPT)])

    run = pl.kernel(
        body,
        out_type=jax.ShapeDtypeStruct((_NC, _NPAD, _D), jnp.float32),
        mesh=mesh,
        scratch_types=scratch,
    )
    if with_e:
        return run(x, e, idx4, zrows)
    return run(x, idx4, zrows)


def _tc_layer(x, acc, W, b, slope):
    bn = 2000

    def body(x_ref, a_ref, w_ref, b_ref, o_ref):
        t = x_ref[...] + a_ref[0] + a_ref[1]
        y = jnp.dot(t, w_ref[...], preferred_element_type=jnp.float32)
        y = y + b_ref[...]
        if slope is not None:
            y = jnp.where(y >= 0, y, slope * y)
        o_ref[...] = y

    return pl.pallas_call(
        body,
        grid=(_N // bn,),
        in_specs=[
            pl.BlockSpec((bn, _D), lambda i: (i, 0)),
            pl.BlockSpec((_NC, bn, _D), lambda i: (0, i, 0)),
            pl.BlockSpec((_D, _D), lambda i: (0, 0)),
            pl.BlockSpec((1, _D), lambda i: (0, 0)),
        ],
        out_specs=pl.BlockSpec((bn, _D), lambda i: (i, 0)),
        out_shape=jax.ShapeDtypeStruct((_N, _D), jnp.float32),
    )(x, acc, W, b.reshape(1, _D))


def _tc_layer_fc(x, acc, W1, b1, W2, b2):
    bn = 2000

    def body(x_ref, a_ref, w1_ref, b1_ref, w2_ref, b2_ref, o_ref):
        t = x_ref[...] + a_ref[0] + a_ref[1]
        y = jnp.dot(t, w1_ref[...], preferred_element_type=jnp.float32)
        y = y + b1_ref[...]
        y = jnp.dot(y, w2_ref[...], preferred_element_type=jnp.float32)
        o_ref[...] = y + b2_ref[...]

    return pl.pallas_call(
        body,
        grid=(_N // bn,),
        in_specs=[
            pl.BlockSpec((bn, _D), lambda i: (i, 0)),
            pl.BlockSpec((_NC, bn, _D), lambda i: (0, i, 0)),
            pl.BlockSpec((_D, _D), lambda i: (0, 0)),
            pl.BlockSpec((1, _D), lambda i: (0, 0)),
            pl.BlockSpec((_D, _D), lambda i: (0, 0)),
            pl.BlockSpec((1, _D), lambda i: (0, 0)),
        ],
        out_specs=pl.BlockSpec((bn, _D), lambda i: (i, 0)),
        out_shape=jax.ShapeDtypeStruct((_N, _D), jnp.float32),
    )(x, acc, W1, b1.reshape(1, _D), W2, b2.reshape(1, _D))


def kernel(n_feat_geo, nfeat_act, efeat_act, edge_index,
           W_geo1, b_geo1, W_geo2, b_geo2,
           W_act1, b_act1, W_act2, b_act2, W_fc, b_fc):
    idx4 = jnp.stack(
        [edge_index[0].reshape(_NW, _NCHUNK, _CH),
         edge_index[1].reshape(_NW, _NCHUNK, _CH)], axis=2)
    zrows = jnp.zeros((_RPT, _D), jnp.float32)

    agg_g = _sc_round(n_feat_geo, None, idx4, zrows)
    agg_a = _sc_round(nfeat_act, efeat_act, idx4, zrows)
    h2 = _tc_layer(n_feat_geo, agg_g, W_geo1, b_geo1, 0.01)
    h1 = _tc_layer_fc(nfeat_act, agg_a, W_act1, b_act1, W_fc, b_fc)
    agg_g = _sc_round(h2, None, idx4, zrows)
    agg_a = _sc_round(h1, efeat_act, idx4, zrows)
    h2 = _tc_layer(h2, agg_g, W_geo2, b_geo2, 0.01)
    h1 = _tc_layer(h1, agg_a, W_act2, b_act2, None)

    return jnp.concatenate([h1, h2], axis=1)


# speedup vs baseline: 1.0725x; 1.0725x over previous
import functools

import jax
import jax.numpy as jnp
from jax import lax
from jax.experimental import pallas as pl
from jax.experimental.pallas import tpu as pltpu
from jax.experimental.pallas import tpu_sc as plsc

_N = 10000
_E = 320000
_D = 128
_NC = 2
_NS = 16
_NW = _NC * _NS
_EPW = _E // _NW
_CH = 80
_NCHUNK = _EPW // _CH
_NPAD = 10112
_RPT = _NPAD // _NS
_VPR = _D // 16


def _sc_gin_round(x, idx4, zrows):
    mesh = plsc.VectorSubcoreMesh(
        core_axis_name="c", subcore_axis_name="s",
        num_cores=_NC, num_subcores=_NS)

    scratch = [
        [pltpu.VMEM((2, _CH), jnp.int32)] * 4,
        [pltpu.VMEM((_CH, _D), jnp.float32)] * 4,
        pltpu.VMEM_SHARED((_NPAD, _D), jnp.float32),
        [pltpu.SemaphoreType.DMA] * 4,
        [pltpu.SemaphoreType.DMA] * 4,
        [pltpu.SemaphoreType.DMA] * 4,
    ]

    def body(x_hbm, idx_hbm, z_hbm, out_hbm, ibuf, rbuf, acc,
             isem, gsem, ssem):
        c = lax.axis_index("c")
        s = lax.axis_index("s")
        wid = c * _NS + s

        pltpu.sync_copy(z_hbm, acc.at[pl.ds(s * _RPT, _RPT)])
        plsc.subcore_barrier()

        def start_idx(j, q):
            pltpu.async_copy(idx_hbm.at[wid, j], ibuf[q % 4], isem[q % 4])

        def wait_idx(q):
            pltpu.make_async_copy(
                idx_hbm.at[wid, 0], ibuf[q % 4], isem[q % 4]).wait()

        def start_gather(q):
            pltpu.async_copy(
                x_hbm.at[ibuf[q % 4].at[0]], rbuf[q % 4], gsem[q % 4])

        def wait_gather(q):
            pltpu.make_async_copy(
                x_hbm.at[ibuf[q % 4].at[0]], rbuf[q % 4],
                gsem[q % 4]).wait()

        def start_scatter(q):
            pltpu.async_copy(rbuf[q % 4], acc.at[ibuf[q % 4].at[1]],
                             ssem[q % 4], add=True)

        def wait_scatter(q):
            pltpu.make_async_copy(rbuf[q % 4], acc.at[ibuf[q % 4].at[1]],
                                  ssem[q % 4]).wait()

        def step(j, q, wait_sc=True, g2=True, i3=True):
            if wait_sc:
                wait_scatter(q + 3)
            if g2:
                wait_idx(q + 2)
                start_gather(q + 2)
            if i3:
                start_idx(j + 3, q + 3)
            wait_gather(q)
            start_scatter(q)

        pltpu.sync_copy(idx_hbm.at[wid, 0], ibuf[0])
        start_idx(1, 1)
        start_idx(2, 2)
        start_gather(0)
        wait_idx(1)
        start_gather(1)
        step(0, 0, wait_sc=False)

        def group(k, carry):
            j0 = 4 * k + 1
            for q in range(4):
                step(j0 + q, (1 + q) % 4)
            return carry
        lax.fori_loop(0, 30, group, 0)

        step(121, 1)
        step(122, 2, i3=False)
        step(123, 3, g2=False, i3=False)
        step(124, 0, g2=False, i3=False)
        wait_scatter(0)
        plsc.subcore_barrier()

        pltpu.sync_copy(acc.at[pl.ds(s * _RPT, _RPT)],
                        out_hbm.at[c, pl.ds(s * _RPT, _RPT)])

    run = pl.kernel(
        body,
        out_type=jax.ShapeDtypeStruct((_NC, _NPAD, _D), jnp.float32),
        mesh=mesh,
        scratch_types=scratch,
    )
    return run(x, idx4, zrows)


def _sc_round(x, e, idx4, zrows):
    with_e = e is not None
    mesh = plsc.VectorSubcoreMesh(
        core_axis_name="c", subcore_axis_name="s",
        num_cores=_NC, num_subcores=_NS)

    scratch = [
        [pltpu.VMEM((2, _CH), jnp.int32)] * 3,
        [pltpu.VMEM((_CH, _D), jnp.float32)] * 2,
        pltpu.VMEM_SHARED((_NPAD, _D), jnp.float32),
        [pltpu.SemaphoreType.DMA] * 3,
        [pltpu.SemaphoreType.DMA] * 2,
        [pltpu.SemaphoreType.DMA] * 2,
    ]
    if with_e:
        scratch.insert(2, [pltpu.VMEM((_CH, _D), jnp.float32)] * 2)
        scratch.append([pltpu.SemaphoreType.DMA] * 2)

    def body(*refs):
        if with_e:
            (x_hbm, e_hbm, idx_hbm, z_hbm, out_hbm,
             ibuf, rbuf, ebuf, acc, isem, gsem, ssem, esem) = refs
        else:
            (x_hbm, idx_hbm, z_hbm, out_hbm,
             ibuf, rbuf, acc, isem, gsem, ssem) = refs
            e_hbm = ebuf = esem = None
        c = lax.axis_index("c")
        s = lax.axis_index("s")
        wid = c * _NS + s

        pltpu.sync_copy(z_hbm, acc.at[pl.ds(s * _RPT, _RPT)])
        plsc.subcore_barrier()

        def start_idx(j, q):
            pltpu.async_copy(idx_hbm.at[wid, j], ibuf[q % 3], isem[q % 3])

        def wait_idx(q):
            pltpu.make_async_copy(
                idx_hbm.at[wid, 0], ibuf[q % 3], isem[q % 3]).wait()

        def start_gather(j, q):
            pltpu.async_copy(
                x_hbm.at[ibuf[q % 3].at[0]], rbuf[q % 2], gsem[q % 2])
            if with_e:
                off = wid * _EPW + j * _CH
                pltpu.async_copy(
                    e_hbm.at[pl.ds(off, _CH)], ebuf[q % 2], esem[q % 2])

        def step(j, q, start_next=True, start_idx2=True):
            if start_next:
                wait_idx(q + 1)
                start_gather(j + 1, q + 1)
            if start_idx2:
                start_idx(j + 2, q + 2)
            pltpu.make_async_copy(
                x_hbm.at[ibuf[q % 3].at[0]], rbuf[q % 2],
                gsem[q % 2]).wait()
            if with_e:
                pltpu.make_async_copy(
                    e_hbm.at[pl.ds(0, _CH)], ebuf[q % 2],
                    esem[q % 2]).wait()

                def rloop(i, c2):
                    for jj in range(_VPR):
                        sl = pl.ds(jj * 16, 16)
                        v = rbuf[q % 2][i, sl] + ebuf[q % 2][i, sl]
                        rbuf[q % 2][i, sl] = jnp.maximum(v, 0.0)
                    return c2
                lax.fori_loop(0, _CH, rloop, 0)
            pltpu.async_copy(rbuf[q % 2], acc.at[ibuf[q % 3].at[1]],
                             ssem[q % 2], add=True)
            pltpu.make_async_copy(rbuf[q % 2], acc.at[ibuf[q % 3].at[1]],
                                  ssem[q % 2]).wait()

        pltpu.sync_copy(idx_hbm.at[wid, 0], ibuf[0])
        start_gather(0, 0)
        start_idx(1, 1)
        step(0, 0)
        step(1, 1)

        def group(k, carry):
            j0 = 6 * k + 2
            for q in range(6):
                step(j0 + q, (2 + q) % 6)
            return carry
        lax.fori_loop(0, 20, group, 0)

        step(122, 2)
        step(123, 3, start_idx2=False)
        step(124, 4, start_next=False, start_idx2=False)
        plsc.subcore_barrier()

        pltpu.sync_copy(acc.at[pl.ds(s * _RPT, _RPT)],
                        out_hbm.at[c, pl.ds(s * _RPT, _RPT)])

    run = pl.kernel(
        body,
        out_type=jax.ShapeDtypeStruct((_NC, _NPAD, _D), jnp.float32),
        mesh=mesh,
        scratch_types=scratch,
    )
    if with_e:
        return run(x, e, idx4, zrows)
    return run(x, idx4, zrows)


def _tc_layer(x, acc, W, b, slope):
    bn = 2000

    def body(x_ref, a_ref, w_ref, b_ref, o_ref):
        t = x_ref[...] + a_ref[0] + a_ref[1]
        y = jnp.dot(t, w_ref[...], preferred_element_type=jnp.float32)
        y = y + b_ref[...]
        if slope is not None:
            y = jnp.where(y >= 0, y, slope * y)
        o_ref[...] = y

    return pl.pallas_call(
        body,
        grid=(_N // bn,),
        in_specs=[
            pl.BlockSpec((bn, _D), lambda i: (i, 0)),
            pl.BlockSpec((_NC, bn, _D), lambda i: (0, i, 0)),
            pl.BlockSpec((_D, _D), lambda i: (0, 0)),
            pl.BlockSpec((1, _D), lambda i: (0, 0)),
        ],
        out_specs=pl.BlockSpec((bn, _D), lambda i: (i, 0)),
        out_shape=jax.ShapeDtypeStruct((_N, _D), jnp.float32),
    )(x, acc, W, b.reshape(1, _D))


def _tc_layer_fc(x, acc, W1, b1, W2, b2):
    bn = 2000

    def body(x_ref, a_ref, w1_ref, b1_ref, w2_ref, b2_ref, o_ref):
        t = x_ref[...] + a_ref[0] + a_ref[1]
        y = jnp.dot(t, w1_ref[...], preferred_element_type=jnp.float32)
        y = y + b1_ref[...]
        y = jnp.dot(y, w2_ref[...], preferred_element_type=jnp.float32)
        o_ref[...] = y + b2_ref[...]

    return pl.pallas_call(
        body,
        grid=(_N // bn,),
        in_specs=[
            pl.BlockSpec((bn, _D), lambda i: (i, 0)),
            pl.BlockSpec((_NC, bn, _D), lambda i: (0, i, 0)),
            pl.BlockSpec((_D, _D), lambda i: (0, 0)),
            pl.BlockSpec((1, _D), lambda i: (0, 0)),
            pl.BlockSpec((_D, _D), lambda i: (0, 0)),
            pl.BlockSpec((1, _D), lambda i: (0, 0)),
        ],
        out_specs=pl.BlockSpec((bn, _D), lambda i: (i, 0)),
        out_shape=jax.ShapeDtypeStruct((_N, _D), jnp.float32),
    )(x, acc, W1, b1.reshape(1, _D), W2, b2.reshape(1, _D))


def kernel(n_feat_geo, nfeat_act, efeat_act, edge_index,
           W_geo1, b_geo1, W_geo2, b_geo2,
           W_act1, b_act1, W_act2, b_act2, W_fc, b_fc):
    idx4 = jnp.stack(
        [edge_index[0].reshape(_NW, _NCHUNK, _CH),
         edge_index[1].reshape(_NW, _NCHUNK, _CH)], axis=2)
    zrows = jnp.zeros((_RPT, _D), jnp.float32)

    agg_g = _sc_gin_round(n_feat_geo, idx4, zrows)
    agg_a = _sc_round(nfeat_act, efeat_act, idx4, zrows)
    h2 = _tc_layer(n_feat_geo, agg_g, W_geo1, b_geo1, 0.01)
    h1 = _tc_layer_fc(nfeat_act, agg_a, W_act1, b_act1, W_fc, b_fc)
    agg_g = _sc_gin_round(h2, idx4, zrows)
    agg_a = _sc_round(h1, efeat_act, idx4, zrows)
    h2 = _tc_layer(h2, agg_g, W_geo2, b_geo2, 0.01)
    h1 = _tc_layer(h1, agg_a, W_act2, b_act2, None)

    return jnp.concatenate([h1, h2], axis=1)
